# Initial kernel scaffold; baseline (speedup 1.0000x reference)
import functools

import jax
import jax.numpy as jnp
import numpy as np
from jax import lax
from jax.experimental import pallas as pl
from jax.experimental.pallas import tpu as pltpu
from jax.experimental.pallas import tpu_sc as plsc

PS = 7
B, HD, T, C, H, W = 1, 4, 2, 32, 32, 32
K = 10
BH = B * HD
PADW = W + PS - 1
ROWW = PADW * C
VWORDS = T * PADW * ROWW
STRIP_ROWS = 14
STRIP = STRIP_ROWS * ROWW
NTEC = 32
QPT = 256


def _sc_body(vpad_hbm, bq_hbm, dq_hbm, out_hbm, vpad_v, bq_v, dq_v, opad_v):
    cid = lax.axis_index("c")
    sid = lax.axis_index("s")
    wid = sid * 2 + cid
    bh = wid // 8
    pltpu.sync_copy(vpad_hbm.at[bh], vpad_v)
    pltpu.sync_copy(bq_hbm.at[wid], bq_v)
    pltpu.sync_copy(dq_hbm.at[wid], dq_v)

    iota = jnp.arange(16, dtype=jnp.int32)
    zeros = jnp.zeros((16,), jnp.float32)

    def zbody(i, _):
        plsc.store_scatter(opad_v, [i * 16 + iota], zeros)
        return 0

    lax.fori_loop(0, STRIP // 16, zbody, 0)

    def qbody(q, _):
        nh = q // 32
        nw = q - nh * 32
        obase = nh * ROWW + nw * C
        bs = []
        ws = []
        for k in range(K):
            qk = jnp.full((16,), q * K + k, jnp.int32)
            bs.append(plsc.load_gather(bq_v, [qk]) + iota)
            ws.append(plsc.load_gather(dq_v, [qk]))

        def ibody(i, _):
            roff = i * ROWW

            def mbody(m, _):
                off = roff + m * 16
                acc = plsc.load_gather(vpad_v, [bs[0] + off]) * ws[0]
                for k in range(1, K):
                    acc = acc + plsc.load_gather(vpad_v, [bs[k] + off]) * ws[k]
                plsc.addupdate_scatter(opad_v, [(obase + off) + iota], acc)
                return 0

            return lax.fori_loop(0, PS * C // 16, mbody, 0)

        lax.fori_loop(0, PS, ibody, 0)
        return 0

    lax.fori_loop(0, QPT, qbody, 0)
    pltpu.sync_copy(opad_v, out_hbm.at[wid])


_sc_call = functools.partial(
    pl.kernel,
    out_type=jax.ShapeDtypeStruct((NTEC, STRIP), jnp.float32),
    mesh=plsc.VectorSubcoreMesh(core_axis_name="c", subcore_axis_name="s"),
    scratch_types=[
        pltpu.VMEM((VWORDS,), jnp.float32),
        pltpu.VMEM((QPT * K,), jnp.int32),
        pltpu.VMEM((QPT * K,), jnp.float32),
        pltpu.VMEM((STRIP,), jnp.float32),
    ],
)(_sc_body)


def _tc_body(parts_ref, invc_ref, out_ref):
    p = parts_ref[...]
    opad = jnp.zeros((8, PADW, ROWW), jnp.float32)
    for blk in range(4):
        opad = opad.at[:, blk * 8: blk * 8 + STRIP_ROWS, :].add(p[:, blk])
    top = opad[:, :H, :]
    for m in range(PS - 1):
        top = top.at[:, 30 - m: 31 - m, :].add(opad[:, 32 + m: 33 + m, :])
    res = top[:, :, : W * C]
    for m in range(PS - 1):
        res = res.at[:, :, (30 - m) * C: (31 - m) * C].add(
            top[:, :, (32 + m) * C: (33 + m) * C]
        )
    out_ref[...] = res * invc_ref[...][None, :, :]


def _fold_counts():
    dy = np.arange(PS)
    hh = np.abs(np.arange(H)[:, None] + dy[None, :])
    hh = np.where(hh > H - 1, 2 * (H - 1) - hh, hh)
    ch = np.zeros(H, np.float64)
    np.add.at(ch, hh, 1.0)
    invc = 1.0 / (ch[:, None] * ch[None, :])
    return np.repeat(invc, C, axis=1).astype(np.float32)


_INVC = _fold_counts()


def kernel(vid_in, dists, inds):
    vid = vid_in.reshape(BH, T, C, H, W).transpose(0, 1, 3, 4, 2)
    vpad = jnp.pad(vid, ((0, 0), (0, 0), (0, PS - 1), (0, PS - 1), (0, 0)),
                   mode="reflect")
    vflat = vpad.reshape(BH, VWORDS)

    ix = inds.reshape(BH, T, H, W, K, 3).astype(jnp.int32)
    tn = ix[..., 0] % T
    bq = ((tn * PADW + ix[..., 1]) * PADW + ix[..., 2]) * C
    bq_t = bq.reshape(BH, T, 4, 8, W, K).reshape(NTEC, QPT * K)
    dq_t = dists.astype(jnp.float32).reshape(BH, T, 4, 8, W, K).reshape(
        NTEC, QPT * K)

    parts = _sc_call(vflat, bq_t, dq_t)
    parts = parts.reshape(8, 4, STRIP_ROWS, ROWW)

    res = pl.pallas_call(
        _tc_body,
        out_shape=jax.ShapeDtypeStruct((8, H, W * C), jnp.float32),
    )(parts, jnp.asarray(_INVC))

    out = res.reshape(BH, T, H, W, C).transpose(0, 1, 4, 2, 3)
    return out.reshape(B, HD, T, C, H, W)


# speedup vs baseline: 123.4644x; 123.4644x over previous
import functools

import jax
import jax.numpy as jnp
import numpy as np
from jax import lax
from jax.experimental import pallas as pl
from jax.experimental.pallas import tpu as pltpu
from jax.experimental.pallas import tpu_sc as plsc

PS = 7
B, HD, T, C, H, W = 1, 4, 2, 32, 32, 32
K = 10
BH = B * HD
PADW = W + PS - 1
ROWW = PADW * C
VWORDS = T * PADW * ROWW
STRIP_ROWS = 14
STRIP = STRIP_ROWS * ROWW
NTEC = 32
QPT = 256


def _sc_body(vpad_hbm, bq_hbm, dq_hbm, out_hbm, vpad_v, bq_v, dq_v, opad_v):
    cid = lax.axis_index("c")
    sid = lax.axis_index("s")
    wid = sid * 2 + cid
    bh = wid // 8
    pltpu.sync_copy(vpad_hbm.at[bh], vpad_v)
    pltpu.sync_copy(bq_hbm.at[wid], bq_v)
    pltpu.sync_copy(dq_hbm.at[wid], dq_v)

    iota = jnp.arange(16, dtype=jnp.int32)
    zeros = jnp.zeros((16,), jnp.float32)

    def zbody(i, _):
        plsc.store_scatter(opad_v, [i * 16 + iota], zeros)
        return 0

    lax.fori_loop(0, STRIP // 16, zbody, 0)

    def qbody(q, _):
        nh = q // 32
        nw = q - nh * 32
        obase = nh * ROWW + nw * C
        bs = []
        ws = []
        for k in range(K):
            qk = jnp.full((16,), q * K + k, jnp.int32)
            bs.append(plsc.load_gather(bq_v, [qk]) + iota)
            ws.append(plsc.load_gather(dq_v, [qk]))

        def ibody(i, _):
            roff = i * ROWW

            def mbody(m, _):
                off = roff + m * 16
                acc = plsc.load_gather(vpad_v, [bs[0] + off]) * ws[0]
                for k in range(1, K):
                    acc = acc + plsc.load_gather(vpad_v, [bs[k] + off]) * ws[k]
                plsc.addupdate_scatter(opad_v, [(obase + off) + iota], acc)
                return 0

            return lax.fori_loop(0, PS * C // 16, mbody, 0)

        lax.fori_loop(0, PS, ibody, 0)
        return 0

    lax.fori_loop(0, QPT, qbody, 0)
    pltpu.sync_copy(opad_v, out_hbm.at[wid])


_sc_call = functools.partial(
    pl.kernel,
    out_type=jax.ShapeDtypeStruct((NTEC, STRIP), jnp.float32),
    mesh=plsc.VectorSubcoreMesh(core_axis_name="c", subcore_axis_name="s"),
    compiler_params=pltpu.CompilerParams(needs_layout_passes=False),
    scratch_types=[
        pltpu.VMEM((VWORDS,), jnp.float32),
        pltpu.VMEM((QPT * K,), jnp.int32),
        pltpu.VMEM((QPT * K,), jnp.float32),
        pltpu.VMEM((STRIP,), jnp.float32),
    ],
)(_sc_body)


def _tc_body(parts_ref, invc_ref, out_ref):
    p = parts_ref[...]

    def pad_rows(x, lo, hi):
        z = jnp.zeros((8, 1, ROWW), jnp.float32)
        pieces = [z] * lo + [x] + [z] * hi
        return jnp.concatenate(pieces, axis=1) if len(pieces) > 1 else x

    opad = sum(
        pad_rows(p[:, blk], blk * 8, PADW - STRIP_ROWS - blk * 8)
        for blk in range(4)
    )
    rev_rows = jnp.concatenate(
        [opad[:, PADW - 1 - m: PADW - m, :] for m in range(PS - 1)], axis=1
    )
    top = opad[:, :H, :] + pad_rows(rev_rows, 25, 1)
    zc = jnp.zeros((8, H, C), jnp.float32)
    rev_cols = [zc] * 25 + [
        top[:, :, (PADW - 1 - m) * C: (PADW - m) * C] for m in range(PS - 1)
    ] + [zc]
    res = top[:, :, : W * C] + jnp.concatenate(rev_cols, axis=2)
    out_ref[...] = res * invc_ref[...][None, :, :]


def _fold_counts():
    dy = np.arange(PS)
    hh = np.abs(np.arange(H)[:, None] + dy[None, :])
    hh = np.where(hh > H - 1, 2 * (H - 1) - hh, hh)
    ch = np.zeros(H, np.float64)
    np.add.at(ch, hh, 1.0)
    invc = 1.0 / (ch[:, None] * ch[None, :])
    return np.repeat(invc, C, axis=1).astype(np.float32)


_INVC = _fold_counts()


def kernel(vid_in, dists, inds):
    vid = vid_in.reshape(BH, T, C, H, W).transpose(0, 1, 3, 4, 2)
    vpad = jnp.pad(vid, ((0, 0), (0, 0), (0, PS - 1), (0, PS - 1), (0, 0)),
                   mode="reflect")
    vflat = vpad.reshape(BH, VWORDS)

    ix = inds.reshape(BH, T, H, W, K, 3).astype(jnp.int32)
    tn = ix[..., 0] % T
    bq = ((tn * PADW + ix[..., 1]) * PADW + ix[..., 2]) * C
    bq_t = bq.reshape(BH, T, 4, 8, W, K).reshape(NTEC, QPT * K)
    dq_t = dists.astype(jnp.float32).reshape(BH, T, 4, 8, W, K).reshape(
        NTEC, QPT * K)

    parts = _sc_call(vflat, bq_t, dq_t)
    parts = parts.reshape(8, 4, STRIP_ROWS, ROWW)

    res = pl.pallas_call(
        _tc_body,
        out_shape=jax.ShapeDtypeStruct((8, H, W * C), jnp.float32),
    )(parts, jnp.asarray(_INVC))

    out = res.reshape(BH, T, H, W, C).transpose(0, 1, 4, 2, 3)
    return out.reshape(B, HD, T, C, H, W)


# speedup vs baseline: 152.0875x; 1.2318x over previous
import functools

import jax
import jax.numpy as jnp
import numpy as np
from jax import lax
from jax.experimental import pallas as pl
from jax.experimental.pallas import tpu as pltpu
from jax.experimental.pallas import tpu_sc as plsc

PS = 7
B, HD, T, C, H, W = 1, 4, 2, 32, 32, 32
K = 10
BH = B * HD
PADW = W + PS - 1
ROWW = PADW * C
VWORDS = T * PADW * ROWW
STRIP_ROWS = 14
STRIP = STRIP_ROWS * ROWW
NTEC = 32
QPT = 256


def _sc_body(vpad_hbm, bq_hbm, dq_hbm, out_hbm, vpad_v, bq_v, dq_v, opad_v):
    cid = lax.axis_index("c")
    sid = lax.axis_index("s")
    wid = sid * 2 + cid
    bh = wid // 8
    pltpu.sync_copy(vpad_hbm.at[bh], vpad_v)
    pltpu.sync_copy(bq_hbm.at[wid], bq_v)
    pltpu.sync_copy(dq_hbm.at[wid], dq_v)

    iota = jnp.arange(16, dtype=jnp.int32)
    zeros = jnp.zeros((16,), jnp.float32)

    def zbody(i, _):
        plsc.store_scatter(opad_v, [i * 16 + iota], zeros)
        return 0

    lax.fori_loop(0, STRIP // 16, zbody, 0)

    def qbody(q, _):
        nh = q // 32
        nw = q - nh * 32
        obase = nh * ROWW + nw * C
        bs = []
        ws = []
        for k in range(K):
            qk = jnp.full((16,), q * K + k, jnp.int32)
            bs.append(plsc.load_gather(bq_v, [qk]) + iota)
            ws.append(plsc.load_gather(dq_v, [qk]))

        def ibody(i, _):
            roff = i * ROWW
            for m in range(PS * C // 16):
                off = roff + m * 16
                acc0 = plsc.load_gather(vpad_v, [bs[0] + off]) * ws[0]
                acc1 = plsc.load_gather(vpad_v, [bs[1] + off]) * ws[1]
                for k in range(2, K, 2):
                    acc0 = acc0 + plsc.load_gather(vpad_v, [bs[k] + off]) * ws[k]
                    acc1 = acc1 + plsc.load_gather(
                        vpad_v, [bs[k + 1] + off]) * ws[k + 1]
                plsc.addupdate_scatter(
                    opad_v, [(obase + off) + iota], acc0 + acc1)
            return 0

        lax.fori_loop(0, PS, ibody, 0)
        return 0

    lax.fori_loop(0, QPT, qbody, 0)
    pltpu.sync_copy(opad_v, out_hbm.at[wid])


_sc_call = functools.partial(
    pl.kernel,
    out_type=jax.ShapeDtypeStruct((NTEC, STRIP), jnp.float32),
    mesh=plsc.VectorSubcoreMesh(core_axis_name="c", subcore_axis_name="s"),
    compiler_params=pltpu.CompilerParams(needs_layout_passes=False),
    scratch_types=[
        pltpu.VMEM((VWORDS,), jnp.float32),
        pltpu.VMEM((QPT * K,), jnp.int32),
        pltpu.VMEM((QPT * K,), jnp.float32),
        pltpu.VMEM((STRIP,), jnp.float32),
    ],
)(_sc_body)


def _tc_body(parts_ref, invc_ref, out_ref):
    p = parts_ref[...]

    def pad_rows(x, lo, hi):
        z = jnp.zeros((8, 1, ROWW), jnp.float32)
        pieces = [z] * lo + [x] + [z] * hi
        return jnp.concatenate(pieces, axis=1) if len(pieces) > 1 else x

    opad = sum(
        pad_rows(p[:, blk], blk * 8, PADW - STRIP_ROWS - blk * 8)
        for blk in range(4)
    )
    rev_rows = jnp.concatenate(
        [opad[:, PADW - 1 - m: PADW - m, :] for m in range(PS - 1)], axis=1
    )
    top = opad[:, :H, :] + pad_rows(rev_rows, 25, 1)
    zc = jnp.zeros((8, H, C), jnp.float32)
    rev_cols = [zc] * 25 + [
        top[:, :, (PADW - 1 - m) * C: (PADW - m) * C] for m in range(PS - 1)
    ] + [zc]
    res = top[:, :, : W * C] + jnp.concatenate(rev_cols, axis=2)
    out_ref[...] = res * invc_ref[...][None, :, :]


def _fold_counts():
    dy = np.arange(PS)
    hh = np.abs(np.arange(H)[:, None] + dy[None, :])
    hh = np.where(hh > H - 1, 2 * (H - 1) - hh, hh)
    ch = np.zeros(H, np.float64)
    np.add.at(ch, hh, 1.0)
    invc = 1.0 / (ch[:, None] * ch[None, :])
    return np.repeat(invc, C, axis=1).astype(np.float32)


_INVC = _fold_counts()


def kernel(vid_in, dists, inds):
    vid = vid_in.reshape(BH, T, C, H, W).transpose(0, 1, 3, 4, 2)
    vpad = jnp.pad(vid, ((0, 0), (0, 0), (0, PS - 1), (0, PS - 1), (0, 0)),
                   mode="reflect")
    vflat = vpad.reshape(BH, VWORDS)

    ix = inds.reshape(BH, T, H, W, K, 3).astype(jnp.int32)
    tn = ix[..., 0] % T
    bq = ((tn * PADW + ix[..., 1]) * PADW + ix[..., 2]) * C
    bq_t = bq.reshape(BH, T, 4, 8, W, K).reshape(NTEC, QPT * K)
    dq_t = dists.astype(jnp.float32).reshape(BH, T, 4, 8, W, K).reshape(
        NTEC, QPT * K)

    parts = _sc_call(vflat, bq_t, dq_t)
    parts = parts.reshape(8, 4, STRIP_ROWS, ROWW)

    res = pl.pallas_call(
        _tc_body,
        out_shape=jax.ShapeDtypeStruct((8, H, W * C), jnp.float32),
    )(parts, jnp.asarray(_INVC))

    out = res.reshape(BH, T, H, W, C).transpose(0, 1, 4, 2, 3)
    return out.reshape(B, HD, T, C, H, W)


# speedup vs baseline: 153.8073x; 1.0113x over previous
import functools

import jax
import jax.numpy as jnp
import numpy as np
from jax import lax
from jax.experimental import pallas as pl
from jax.experimental.pallas import tpu as pltpu
from jax.experimental.pallas import tpu_sc as plsc

PS = 7
B, HD, T, C, H, W = 1, 4, 2, 32, 32, 32
K = 10
BH = B * HD
PADW = W + PS - 1
ROWW = PADW * C
VWORDS = T * PADW * ROWW
STRIP_ROWS = 14
STRIP = STRIP_ROWS * ROWW
NTEC = 32
QPT = 256


def _sc_body(vpad_hbm, bq_hbm, dq_hbm, out_hbm, vpad_v, bq_v, dq_v, opad_v):
    cid = lax.axis_index("c")
    sid = lax.axis_index("s")
    wid = sid * 2 + cid
    bh = wid // 8
    pltpu.sync_copy(vpad_hbm.at[bh], vpad_v)
    pltpu.sync_copy(bq_hbm.at[wid], bq_v)
    pltpu.sync_copy(dq_hbm.at[wid], dq_v)

    iota = jnp.arange(16, dtype=jnp.int32)
    zeros = jnp.zeros((16,), jnp.float32)

    def zbody(i, _):
        plsc.store_scatter(opad_v, [i * 16 + iota], zeros)
        return 0

    lax.fori_loop(0, STRIP // 16, zbody, 0)

    def qbody(q, _):
        nh = q // 32
        nw = q - nh * 32
        obase = nh * ROWW + nw * C
        bvec = bq_v[pl.ds(q * K, 16)]
        wvec = dq_v[pl.ds(q * K, 16)]
        bs = [bvec[k] for k in range(K)]
        ws = [wvec[k] for k in range(K)]

        def ibody(i, _):
            roff = i * ROWW
            for m in range(PS * C // 16):
                off = roff + m * 16
                acc0 = vpad_v[pl.ds(bs[0] + off, 16)] * ws[0]
                acc1 = vpad_v[pl.ds(bs[1] + off, 16)] * ws[1]
                for k in range(2, K, 2):
                    acc0 = acc0 + vpad_v[pl.ds(bs[k] + off, 16)] * ws[k]
                    acc1 = acc1 + vpad_v[pl.ds(bs[k + 1] + off, 16)] * ws[k + 1]
                plsc.addupdate(opad_v.at[pl.ds(obase + off, 16)], acc0 + acc1)
            return 0

        lax.fori_loop(0, PS, ibody, 0)
        return 0

    lax.fori_loop(0, QPT, qbody, 0)
    pltpu.sync_copy(opad_v, out_hbm.at[wid])


_sc_call = functools.partial(
    pl.kernel,
    out_type=jax.ShapeDtypeStruct((NTEC, STRIP), jnp.float32),
    mesh=plsc.VectorSubcoreMesh(core_axis_name="c", subcore_axis_name="s"),
    compiler_params=pltpu.CompilerParams(needs_layout_passes=False),
    scratch_types=[
        pltpu.VMEM((VWORDS,), jnp.float32),
        pltpu.VMEM((QPT * K,), jnp.int32),
        pltpu.VMEM((QPT * K,), jnp.float32),
        pltpu.VMEM((STRIP,), jnp.float32),
    ],
)(_sc_body)


def _tc_body(parts_ref, invc_ref, out_ref):
    p = parts_ref[...]

    def pad_rows(x, lo, hi):
        z = jnp.zeros((8, 1, ROWW), jnp.float32)
        pieces = [z] * lo + [x] + [z] * hi
        return jnp.concatenate(pieces, axis=1) if len(pieces) > 1 else x

    opad = sum(
        pad_rows(p[:, blk], blk * 8, PADW - STRIP_ROWS - blk * 8)
        for blk in range(4)
    )
    rev_rows = jnp.concatenate(
        [opad[:, PADW - 1 - m: PADW - m, :] for m in range(PS - 1)], axis=1
    )
    top = opad[:, :H, :] + pad_rows(rev_rows, 25, 1)
    zc = jnp.zeros((8, H, C), jnp.float32)
    rev_cols = [zc] * 25 + [
        top[:, :, (PADW - 1 - m) * C: (PADW - m) * C] for m in range(PS - 1)
    ] + [zc]
    res = top[:, :, : W * C] + jnp.concatenate(rev_cols, axis=2)
    out_ref[...] = res * invc_ref[...][None, :, :]


def _fold_counts():
    dy = np.arange(PS)
    hh = np.abs(np.arange(H)[:, None] + dy[None, :])
    hh = np.where(hh > H - 1, 2 * (H - 1) - hh, hh)
    ch = np.zeros(H, np.float64)
    np.add.at(ch, hh, 1.0)
    invc = 1.0 / (ch[:, None] * ch[None, :])
    return np.repeat(invc, C, axis=1).astype(np.float32)


_INVC = _fold_counts()


def kernel(vid_in, dists, inds):
    vid = vid_in.reshape(BH, T, C, H, W).transpose(0, 1, 3, 4, 2)
    vpad = jnp.pad(vid, ((0, 0), (0, 0), (0, PS - 1), (0, PS - 1), (0, 0)),
                   mode="reflect")
    vflat = vpad.reshape(BH, VWORDS)

    ix = inds.reshape(BH, T, H, W, K, 3).astype(jnp.int32)
    tn = ix[..., 0] % T
    bq = ((tn * PADW + ix[..., 1]) * PADW + ix[..., 2]) * C
    bq_t = bq.reshape(BH, T, 4, 8, W, K).reshape(NTEC, QPT * K)
    dq_t = dists.astype(jnp.float32).reshape(BH, T, 4, 8, W, K).reshape(
        NTEC, QPT * K)

    parts = _sc_call(vflat, bq_t, dq_t)
    parts = parts.reshape(8, 4, STRIP_ROWS, ROWW)

    res = pl.pallas_call(
        _tc_body,
        out_shape=jax.ShapeDtypeStruct((8, H, W * C), jnp.float32),
    )(parts, jnp.asarray(_INVC))

    out = res.reshape(BH, T, H, W, C).transpose(0, 1, 4, 2, 3)
    return out.reshape(B, HD, T, C, H, W)


# speedup vs baseline: 223.5574x; 1.4535x over previous
import functools

import jax
import jax.numpy as jnp
import numpy as np
from jax import lax
from jax.experimental import pallas as pl
from jax.experimental.pallas import tpu as pltpu
from jax.experimental.pallas import tpu_sc as plsc

PS = 7
B, HD, T, C, H, W = 1, 4, 2, 32, 32, 32
K = 10
BH = B * HD
PADW = W + PS - 1
ROWW = PADW * C
VWORDS = T * PADW * ROWW
STRIP_ROWS = 14
STRIP = STRIP_ROWS * ROWW
NTEC = 32
QPT = 256


def _sc_body(vpad_hbm, bq_hbm, dq_hbm, out_hbm, vpad_v, bq_v, dq_v, opad_v):
    cid = lax.axis_index("c")
    sid = lax.axis_index("s")
    wid = sid * 2 + cid
    bh = wid // 8
    pltpu.sync_copy(vpad_hbm.at[bh], vpad_v)
    pltpu.sync_copy(bq_hbm.at[wid], bq_v)
    pltpu.sync_copy(dq_hbm.at[wid], dq_v)

    iota = jnp.arange(16, dtype=jnp.int32)
    zeros = jnp.zeros((16,), jnp.float32)

    def zbody(i, _):
        plsc.store_scatter(opad_v, [i * 16 + iota], zeros)
        return 0

    lax.fori_loop(0, STRIP // 16, zbody, 0)

    def qbody(q, _):
        nh = q // 32
        nw = q - nh * 32
        obase = nh * ROWW + nw * C
        bvec = bq_v[pl.ds(q * K, 16)]
        wvec = dq_v[pl.ds(q * K, 16)]
        bs = [bvec[k] for k in range(K)]
        ws = [wvec[k] for k in range(K)]

        @plsc.parallel_loop(0, PS)
        def ibody(i):
            roff = i * ROWW
            for m in range(0, PS * C // 16, 2):
                offa = roff + m * 16
                offb = offa + 16
                a0 = vpad_v[pl.ds(bs[0] + offa, 16)] * ws[0]
                b0 = vpad_v[pl.ds(bs[0] + offb, 16)] * ws[0]
                a1 = vpad_v[pl.ds(bs[1] + offa, 16)] * ws[1]
                b1 = vpad_v[pl.ds(bs[1] + offb, 16)] * ws[1]
                for k in range(2, K, 2):
                    a0 = a0 + vpad_v[pl.ds(bs[k] + offa, 16)] * ws[k]
                    b0 = b0 + vpad_v[pl.ds(bs[k] + offb, 16)] * ws[k]
                    a1 = a1 + vpad_v[pl.ds(bs[k + 1] + offa, 16)] * ws[k + 1]
                    b1 = b1 + vpad_v[pl.ds(bs[k + 1] + offb, 16)] * ws[k + 1]
                plsc.addupdate(opad_v.at[pl.ds(obase + offa, 16)], a0 + a1)
                plsc.addupdate(opad_v.at[pl.ds(obase + offb, 16)], b0 + b1)

        return 0

    lax.fori_loop(0, QPT, qbody, 0)
    pltpu.sync_copy(opad_v, out_hbm.at[wid])


_sc_call = functools.partial(
    pl.kernel,
    out_type=jax.ShapeDtypeStruct((NTEC, STRIP), jnp.float32),
    mesh=plsc.VectorSubcoreMesh(core_axis_name="c", subcore_axis_name="s"),
    compiler_params=pltpu.CompilerParams(needs_layout_passes=False),
    scratch_types=[
        pltpu.VMEM((VWORDS,), jnp.float32),
        pltpu.VMEM((QPT * K,), jnp.int32),
        pltpu.VMEM((QPT * K,), jnp.float32),
        pltpu.VMEM((STRIP,), jnp.float32),
    ],
)(_sc_body)


def _tc_body(parts_ref, invc_ref, out_ref):
    p = parts_ref[...]

    def pad_rows(x, lo, hi):
        z = jnp.zeros((8, 1, ROWW), jnp.float32)
        pieces = [z] * lo + [x] + [z] * hi
        return jnp.concatenate(pieces, axis=1) if len(pieces) > 1 else x

    opad = sum(
        pad_rows(p[:, blk], blk * 8, PADW - STRIP_ROWS - blk * 8)
        for blk in range(4)
    )
    rev_rows = jnp.concatenate(
        [opad[:, PADW - 1 - m: PADW - m, :] for m in range(PS - 1)], axis=1
    )
    top = opad[:, :H, :] + pad_rows(rev_rows, 25, 1)
    zc = jnp.zeros((8, H, C), jnp.float32)
    rev_cols = [zc] * 25 + [
        top[:, :, (PADW - 1 - m) * C: (PADW - m) * C] for m in range(PS - 1)
    ] + [zc]
    res = top[:, :, : W * C] + jnp.concatenate(rev_cols, axis=2)
    out_ref[...] = res * invc_ref[...][None, :, :]


def _fold_counts():
    dy = np.arange(PS)
    hh = np.abs(np.arange(H)[:, None] + dy[None, :])
    hh = np.where(hh > H - 1, 2 * (H - 1) - hh, hh)
    ch = np.zeros(H, np.float64)
    np.add.at(ch, hh, 1.0)
    invc = 1.0 / (ch[:, None] * ch[None, :])
    return np.repeat(invc, C, axis=1).astype(np.float32)


_INVC = _fold_counts()


def kernel(vid_in, dists, inds):
    vid = vid_in.reshape(BH, T, C, H, W).transpose(0, 1, 3, 4, 2)
    vpad = jnp.pad(vid, ((0, 0), (0, 0), (0, PS - 1), (0, PS - 1), (0, 0)),
                   mode="reflect")
    vflat = vpad.reshape(BH, VWORDS)

    ix = inds.reshape(BH, T, H, W, K, 3).astype(jnp.int32)
    tn = ix[..., 0] % T
    bq = ((tn * PADW + ix[..., 1]) * PADW + ix[..., 2]) * C
    bq_t = bq.reshape(BH, T, 4, 8, W, K).reshape(NTEC, QPT * K)
    dq_t = dists.astype(jnp.float32).reshape(BH, T, 4, 8, W, K).reshape(
        NTEC, QPT * K)

    parts = _sc_call(vflat, bq_t, dq_t)
    parts = parts.reshape(8, 4, STRIP_ROWS, ROWW)

    res = pl.pallas_call(
        _tc_body,
        out_shape=jax.ShapeDtypeStruct((8, H, W * C), jnp.float32),
    )(parts, jnp.asarray(_INVC))

    out = res.reshape(BH, T, H, W, C).transpose(0, 1, 4, 2, 3)
    return out.reshape(B, HD, T, C, H, W)
